# baseline (speedup 1.0000x reference)
import functools

import jax
import jax.numpy as jnp
from jax import lax
from jax.experimental import pallas as pl
from jax.experimental.pallas import tpu as pltpu
from jax.experimental.pallas import tpu_sc as plsc

N = 10000
D = 128
G = 64
L = 3
BN_EPS = 1e-5

NC = 2
NS = 16
NW = NC * NS
CHUNK = 64
BLK = 1000
NBLK = N // BLK
N_ACC = 10240
ROWS_PER_TILE = N_ACC // NS
ZCOPY = 64


RING = 4


def _make_agg(chunks_per_worker: int):
    mesh = plsc.VectorSubcoreMesh(core_axis_name="c", subcore_axis_name="s")
    cpw = chunks_per_worker
    NPHASE = 4
    assert cpw % (4 * NPHASE) == 0
    P = cpw // NPHASE

    @functools.partial(
        pl.kernel,
        out_type=jax.ShapeDtypeStruct((NC, N_ACC, D), jnp.float32),
        mesh=mesh,
        scratch_types=[
            pltpu.VMEM((cpw // NPHASE, 2, CHUNK), jnp.int32),
            pltpu.VMEM((RING, CHUNK, D), jnp.float32),
            pltpu.VMEM_SHARED((N_ACC, D), jnp.float32),
            pltpu.SemaphoreType.DMA,
            [pltpu.SemaphoreType.DMA] * RING,
        ],
    )
    def agg(ztab_hbm, idx_hbm, out_hbm, idx_v, rows, acc_sh, gsem, ssems):
        c = lax.axis_index("c")
        s = lax.axis_index("s")
        wid = s * NC + c

        def zrow(r, carry):
            for k in range(D // 16):
                rows[0, r, pl.ds(k * 16, 16)] = jnp.zeros((16,), jnp.float32)
            return carry
        lax.fori_loop(0, CHUNK, zrow, 0)
        base = s * ROWS_PER_TILE
        for k in range(ROWS_PER_TILE // ZCOPY):
            pltpu.sync_copy(rows.at[0, pl.ds(0, ZCOPY)],
                            acc_sh.at[pl.ds(base + k * ZCOPY, ZCOPY)])
        plsc.subcore_barrier()

        def fire_g(j, b):
            pltpu.async_copy(ztab_hbm.at[idx_v.at[j, 0]], rows.at[b], gsem)

        def fire_s(j, b):
            pltpu.async_copy(rows.at[b], acc_sh.at[idx_v.at[j, 1]], ssems[b],
                             add=True)

        def drain_g():
            pltpu.make_async_copy(ztab_hbm.at[pl.ds(0, CHUNK)], rows.at[0],
                                  gsem).wait()

        def drain_s(b):
            pltpu.make_async_copy(rows.at[b], acc_sh.at[pl.ds(0, CHUNK)],
                                  ssems[b]).wait()

        def step(j, bn, bc, first):
            if not first:
                drain_s(bn)
            fire_g(jnp.minimum(j + 2, P - 1), bn)
            drain_g()
            fire_s(j, bc)

        for p in range(NPHASE):
            pltpu.sync_copy(idx_hbm.at[wid, pl.ds(p * P, P)], idx_v)
            fire_g(jnp.int32(0), 0)
            fire_g(jnp.int32(1), 1)
            step(jnp.int32(0), 2, 0, True)
            step(jnp.int32(1), 3, 1, True)

            def body(u, carry):
                j0 = 2 + 4 * u
                step(j0, 0, 2, False)
                step(j0 + 1, 1, 3, False)
                step(j0 + 2, 2, 0, False)
                step(j0 + 3, 3, 1, False)
                return carry
            lax.fori_loop(0, (P - 4) // 4, body, 0)

            step(jnp.int32(P - 2), 0, 2, False)
            step(jnp.int32(P - 1), 1, 3, False)
            drain_g()
            drain_g()
            drain_s(2)
            drain_s(3)
        plsc.subcore_barrier()

        for k in range(ROWS_PER_TILE // ZCOPY):
            off = base + k * ZCOPY
            pltpu.sync_copy(acc_sh.at[pl.ds(off, ZCOPY)],
                            rows.at[0, pl.ds(0, ZCOPY)])
            pltpu.sync_copy(rows.at[0, pl.ds(0, ZCOPY)],
                            out_hbm.at[c, pl.ds(off, ZCOPY)])

    return agg


def _mlp_body(last: bool, z_ref, parts_ref, bt_ref, w1_ref, b1_ref, w2_ref,
              b2_ref, h_ref, g_ref):
    h = z_ref[...] + parts_ref[0] + parts_ref[1]
    h = jnp.maximum(
        jnp.dot(h, w1_ref[...], preferred_element_type=jnp.float32) + b1_ref[...],
        0.0)
    h = jnp.dot(h, w2_ref[...], preferred_element_type=jnp.float32) + b2_ref[...]
    if not last:
        h = jnp.maximum(h, 0.0)
    h_ref[...] = h

    b = bt_ref[0, 0, :]
    oh_t = (lax.broadcasted_iota(jnp.int32, (G, BLK), 0) == b[None, :]
            ).astype(jnp.float32)
    gpart = jnp.dot(oh_t, h, preferred_element_type=jnp.float32)

    @pl.when(pl.program_id(0) == 0)
    def _():
        g_ref[...] = jnp.zeros_like(g_ref)
    g_ref[...] += gpart


def _make_mlp(last: bool):
    return pl.pallas_call(
        functools.partial(_mlp_body, last),
        grid=(NBLK,),
        in_specs=[
            pl.BlockSpec((BLK, D), lambda i: (i, 0)),
            pl.BlockSpec((NC, BLK, D), lambda i: (0, i, 0)),
            pl.BlockSpec((1, 1, BLK), lambda i: (i, 0, 0)),
            pl.BlockSpec((D, D), lambda i: (0, 0)),
            pl.BlockSpec((1, D), lambda i: (0, 0)),
            pl.BlockSpec((D, D), lambda i: (0, 0)),
            pl.BlockSpec((1, D), lambda i: (0, 0)),
        ],
        out_specs=[
            pl.BlockSpec((BLK, D), lambda i: (i, 0)),
            pl.BlockSpec((G, D), lambda i: (0, 0)),
        ],
        out_shape=[
            jax.ShapeDtypeStruct((N, D), jnp.float32),
            jax.ShapeDtypeStruct((G, D), jnp.float32),
        ],
    )


def kernel(x, edge_index, batch,
           W1_0, b1_0, W2_0, b2_0, gamma_0, beta_0,
           W1_1, b1_1, W2_1, b2_1, gamma_1, beta_1,
           W1_2, b1_2, W2_2, b2_2, gamma_2, beta_2):
    params = [
        (W1_0, b1_0, W2_0, b2_0, gamma_0, beta_0),
        (W1_1, b1_1, W2_1, b2_1, gamma_1, beta_1),
        (W1_2, b1_2, W2_2, b2_2, gamma_2, beta_2),
    ]
    src = edge_index[0]
    dst = edge_index[1]
    e = src.shape[0]
    cpw = -(-e // (NW * CHUNK))
    cpw = -(-cpw // 16) * 16
    e_pad = NW * cpw * CHUNK
    src_p = jnp.concatenate(
        [src, jnp.full((e_pad - e,), N, jnp.int32)]).reshape(NW, cpw, 1, CHUNK)
    dst_p = jnp.concatenate(
        [dst, jnp.zeros((e_pad - e,), jnp.int32)]).reshape(NW, cpw, 1, CHUNK)
    idx_p = jnp.concatenate([src_p, dst_p], axis=2)
    batch3 = batch.reshape(NBLK, 1, BLK)
    zero_row = jnp.zeros((1, D), jnp.float32)

    agg_fn = _make_agg(cpw)
    mlp_mid = _make_mlp(last=False)
    mlp_last = _make_mlp(last=True)

    z = x
    zs, gs = [], []
    for l in range(L):
        W1, b1, W2, b2, gamma, beta = params[l]
        scale = gamma / jnp.sqrt(1.0 + BN_EPS)
        w2f = W2 * scale[None, :]
        b2f = (b2 * scale + beta).reshape(1, D)
        b1r = b1.reshape(1, D)

        ztab = jnp.concatenate([z, zero_row], axis=0)
        parts = agg_fn(ztab, idx_p)
        mlp = mlp_last if l == L - 1 else mlp_mid
        h, g = mlp(z, parts, batch3, W1, b1r, w2f, b2f)
        zs.append(h)
        gs.append(g)
        z = h

    return (jnp.concatenate(zs, axis=1), jnp.concatenate(gs, axis=1))


# speedup vs baseline: 1.8939x; 1.8939x over previous
import functools

import jax
import jax.numpy as jnp
from jax import lax
from jax.experimental import pallas as pl
from jax.experimental.pallas import tpu as pltpu
from jax.experimental.pallas import tpu_sc as plsc

N = 10000
D = 128
G = 64
L = 3
BN_EPS = 1e-5

NC = 2
NS = 16
NW = NC * NS
CHUNK = 128
BLK = 1000
NBLK = N // BLK
N_ACC = 10240
ROWS_PER_TILE = N_ACC // NS
ZCOPY = 128
CORE0_NUM = 2
CORE_DEN = 3


def _make_agg(cpw0: int, cpw1: int):
    mesh = plsc.VectorSubcoreMesh(core_axis_name="c", subcore_axis_name="s")
    cpw_max = max(cpw0, cpw1)

    @functools.partial(
        pl.kernel,
        out_type=jax.ShapeDtypeStruct((NC, N_ACC, D), jnp.float32),
        mesh=mesh,
        scratch_types=[
            pltpu.VMEM((cpw_max, 2, CHUNK), jnp.int32),
            pltpu.VMEM((CHUNK, D), jnp.float32),
            pltpu.VMEM_SHARED((N_ACC, D), jnp.float32),
            pltpu.SemaphoreType.DMA,
        ],
    )
    def agg(ztab_hbm, idx_hbm, out_hbm, idx_v, rows, acc_sh, sem):
        c = lax.axis_index("c")
        s = lax.axis_index("s")
        wid = s * NC + c
        nc = lax.select(c == 0, jnp.int32(cpw0), jnp.int32(cpw1))

        def zrow(r, carry):
            for k in range(D // 16):
                rows[r, pl.ds(k * 16, 16)] = jnp.zeros((16,), jnp.float32)
            return carry
        lax.fori_loop(0, CHUNK, zrow, 0)
        base = s * ROWS_PER_TILE
        for k in range(ROWS_PER_TILE // ZCOPY):
            pltpu.sync_copy(rows.at[pl.ds(0, ZCOPY)],
                            acc_sh.at[pl.ds(base + k * ZCOPY, ZCOPY)])
        plsc.subcore_barrier()

        pltpu.sync_copy(idx_hbm.at[wid], idx_v)

        def body(j, carry):
            pltpu.async_copy(ztab_hbm.at[idx_v.at[j, 0]], rows, sem).wait()
            pltpu.sync_copy(rows, acc_sh.at[idx_v.at[j, 1]], add=True)
            return carry
        lax.fori_loop(0, nc, body, 0)
        plsc.subcore_barrier()

        for k in range(ROWS_PER_TILE // ZCOPY):
            off = base + k * ZCOPY
            pltpu.sync_copy(acc_sh.at[pl.ds(off, ZCOPY)],
                            rows.at[pl.ds(0, ZCOPY)])
            pltpu.sync_copy(rows.at[pl.ds(0, ZCOPY)],
                            out_hbm.at[c, pl.ds(off, ZCOPY)])

    return agg


def _mlp_body(last: bool, z_ref, parts_ref, bt_ref, w1_ref, b1_ref, w2_ref,
              b2_ref, h_ref, g_ref):
    h = z_ref[...] + parts_ref[0] + parts_ref[1]
    h = jnp.maximum(
        jnp.dot(h, w1_ref[...], preferred_element_type=jnp.float32) + b1_ref[...],
        0.0)
    h = jnp.dot(h, w2_ref[...], preferred_element_type=jnp.float32) + b2_ref[...]
    if not last:
        h = jnp.maximum(h, 0.0)
    h_ref[...] = h

    b = bt_ref[0, 0, :]
    oh_t = (lax.broadcasted_iota(jnp.int32, (G, BLK), 0) == b[None, :]
            ).astype(jnp.float32)
    gpart = jnp.dot(oh_t, h, preferred_element_type=jnp.float32)

    @pl.when(pl.program_id(0) == 0)
    def _():
        g_ref[...] = jnp.zeros_like(g_ref)
    g_ref[...] += gpart


def _make_mlp(last: bool):
    return pl.pallas_call(
        functools.partial(_mlp_body, last),
        grid=(NBLK,),
        in_specs=[
            pl.BlockSpec((BLK, D), lambda i: (i, 0)),
            pl.BlockSpec((NC, BLK, D), lambda i: (0, i, 0)),
            pl.BlockSpec((1, 1, BLK), lambda i: (i, 0, 0)),
            pl.BlockSpec((D, D), lambda i: (0, 0)),
            pl.BlockSpec((1, D), lambda i: (0, 0)),
            pl.BlockSpec((D, D), lambda i: (0, 0)),
            pl.BlockSpec((1, D), lambda i: (0, 0)),
        ],
        out_specs=[
            pl.BlockSpec((BLK, D), lambda i: (i, 0)),
            pl.BlockSpec((G, D), lambda i: (0, 0)),
        ],
        out_shape=[
            jax.ShapeDtypeStruct((N, D), jnp.float32),
            jax.ShapeDtypeStruct((G, D), jnp.float32),
        ],
    )


def kernel(x, edge_index, batch,
           W1_0, b1_0, W2_0, b2_0, gamma_0, beta_0,
           W1_1, b1_1, W2_1, b2_1, gamma_1, beta_1,
           W1_2, b1_2, W2_2, b2_2, gamma_2, beta_2):
    params = [
        (W1_0, b1_0, W2_0, b2_0, gamma_0, beta_0),
        (W1_1, b1_1, W2_1, b2_1, gamma_1, beta_1),
        (W1_2, b1_2, W2_2, b2_2, gamma_2, beta_2),
    ]
    src = edge_index[0]
    dst = edge_index[1]
    e = src.shape[0]
    tot = -(-e // (NS * CHUNK))
    cpw0 = tot * CORE0_NUM // CORE_DEN
    cpw1 = tot - cpw0
    e_pad = NS * tot * CHUNK
    src_p = jnp.concatenate([src, jnp.full((e_pad - e,), N, jnp.int32)])
    dst_p = jnp.concatenate([dst, jnp.zeros((e_pad - e,), jnp.int32)])
    split = NS * cpw0 * CHUNK

    def pools(flat):
        p0 = flat[:split].reshape(NS, cpw0, 1, CHUNK)
        p1 = flat[split:].reshape(NS, cpw1, 1, CHUNK)
        p1 = jnp.concatenate(
            [p1, jnp.zeros((NS, cpw0 - cpw1, 1, CHUNK), jnp.int32)], axis=1)
        return p0, p1
    s0, s1 = pools(src_p)
    d0, d1 = pools(dst_p)
    i0 = jnp.concatenate([s0, d0], axis=2)
    i1 = jnp.concatenate([s1, d1], axis=2)
    idx_p = jnp.stack([i0, i1], axis=1).reshape(NW, cpw0, 2, CHUNK)
    batch3 = batch.reshape(NBLK, 1, BLK)
    zero_row = jnp.zeros((1, D), jnp.float32)

    agg_fn = _make_agg(cpw0, cpw1)
    mlp_mid = _make_mlp(last=False)
    mlp_last = _make_mlp(last=True)

    z = x
    zs, gs = [], []
    for l in range(L):
        W1, b1, W2, b2, gamma, beta = params[l]
        scale = gamma / jnp.sqrt(1.0 + BN_EPS)
        w2f = W2 * scale[None, :]
        b2f = (b2 * scale + beta).reshape(1, D)
        b1r = b1.reshape(1, D)

        ztab = jnp.concatenate([z, zero_row], axis=0)
        parts = agg_fn(ztab, idx_p)
        mlp = mlp_last if l == L - 1 else mlp_mid
        h, g = mlp(z, parts, batch3, W1, b1r, w2f, b2f)
        zs.append(h)
        gs.append(g)
        z = h

    return (jnp.concatenate(zs, axis=1), jnp.concatenate(gs, axis=1))
